# Initial kernel scaffold; baseline (speedup 1.0000x reference)
import jax
import jax.numpy as jnp
from jax.experimental import pallas as pl


def kernel(x, entropy, W_gate, W_ent, b_ent, temperature):
    raise NotImplementedError("write your pallas kernel here")


# speedup vs baseline: 1.5619x; 1.5619x over previous
import functools

import jax
import jax.numpy as jnp
from jax import lax
from jax.experimental import pallas as pl
from jax.experimental.pallas import tpu as pltpu
from jax.experimental.pallas import tpu_sc as plsc

_D = 2048
_E = 64
_B = 4
_T = 4096
_TOKENS = _B * _T
_BT = 1024
_LANES = 16


def _gate_body(ent_ref, temp_ref, x_ref, w_ref, went_ref, bent_ref,
               out_ref, outT_ref):
    i = pl.program_id(0)
    b = i // (_T // _BT)
    ent = ent_ref[b, 0]
    denom = jnp.abs(temp_ref[0]) + 1e-6
    logits = lax.dot_general(
        x_ref[...], w_ref[...],
        dimension_numbers=(((1,), (1,)), ((), ())),
        preferred_element_type=jnp.float32,
    )
    bias = ent * went_ref[...] + bent_ref[...]
    res = (logits + bias) / denom
    out_ref[...] = res
    outT_ref[...] = res.T


def _gate_logits(x2, W_gate, went, bent, entropy, temperature):
    grid = (_TOKENS // _BT,)
    return pl.pallas_call(
        _gate_body,
        grid=grid,
        in_specs=[
            pl.BlockSpec(memory_space=pltpu.SMEM),
            pl.BlockSpec(memory_space=pltpu.SMEM),
            pl.BlockSpec((_BT, _D), lambda i: (i, 0)),
            pl.BlockSpec((_E, _D), lambda i: (0, 0)),
            pl.BlockSpec((1, _E), lambda i: (0, 0)),
            pl.BlockSpec((1, _E), lambda i: (0, 0)),
        ],
        out_specs=[
            pl.BlockSpec((_BT, _E), lambda i: (i, 0)),
            pl.BlockSpec((_E, _BT), lambda i: (0, i)),
        ],
        out_shape=[
            jax.ShapeDtypeStruct((_TOKENS, _E), jnp.float32),
            jax.ShapeDtypeStruct((_E, _TOKENS), jnp.float32),
        ],
        compiler_params=pltpu.CompilerParams(
            dimension_semantics=("arbitrary",),
        ),
    )(entropy, temperature, x2, W_gate, went, bent)


def _topk_call(logitsT):
    info = plsc.get_sparse_core_info()
    nc, ns = info.num_cores, info.num_subcores
    nw = nc * ns
    tok_per = _TOKENS // nw
    ngroups = tok_per // _LANES
    mesh = plsc.VectorSubcoreMesh(core_axis_name="c", subcore_axis_name="s")

    def body(lt_hbm, w_hbm, i_hbm, chunk, w1v, w2v, i1v, i2v):
        wid = lax.axis_index("s") * nc + lax.axis_index("c")
        base = wid * tok_per
        pltpu.sync_copy(lt_hbm.at[:, pl.ds(base, tok_per)], chunk)

        def group(g, carry):
            t0 = g * _LANES
            m1 = jnp.full((_LANES,), -jnp.inf, jnp.float32)
            m2 = jnp.full((_LANES,), -jnp.inf, jnp.float32)
            i1 = jnp.zeros((_LANES,), jnp.int32)
            i2 = jnp.zeros((_LANES,), jnp.int32)
            for e in range(_E):
                col = jnp.full((_LANES,), e, jnp.int32)
                v = chunk[e, pl.ds(t0, _LANES)]
                gt1 = v > m1
                gt2 = v > m2
                m2n = jnp.where(gt2, jnp.where(gt1, m1, v), m2)
                i2n = jnp.where(gt2, jnp.where(gt1, i1, col), i2)
                m1 = jnp.where(gt1, v, m1)
                i1 = jnp.where(gt1, col, i1)
                m2, i2 = m2n, i2n
            z = jnp.exp(m2 - m1)
            s = 1.0 + z
            w1v[pl.ds(t0, _LANES)] = 1.0 / s
            w2v[pl.ds(t0, _LANES)] = z / s
            i1v[pl.ds(t0, _LANES)] = i1
            i2v[pl.ds(t0, _LANES)] = i2
            return carry

        lax.fori_loop(0, ngroups, group, 0)
        pltpu.sync_copy(w1v, w_hbm.at[0, pl.ds(base, tok_per)])
        pltpu.sync_copy(w2v, w_hbm.at[1, pl.ds(base, tok_per)])
        pltpu.sync_copy(i1v, i_hbm.at[0, pl.ds(base, tok_per)])
        pltpu.sync_copy(i2v, i_hbm.at[1, pl.ds(base, tok_per)])

    call = functools.partial(
        pl.kernel,
        mesh=mesh,
        out_type=[
            jax.ShapeDtypeStruct((2, _TOKENS), jnp.float32),
            jax.ShapeDtypeStruct((2, _TOKENS), jnp.int32),
        ],
        scratch_types=[
            pltpu.VMEM((_E, tok_per), jnp.float32),
            pltpu.VMEM((tok_per,), jnp.float32),
            pltpu.VMEM((tok_per,), jnp.float32),
            pltpu.VMEM((tok_per,), jnp.int32),
            pltpu.VMEM((tok_per,), jnp.int32),
        ],
    )(body)
    return call(logitsT)


def kernel(x, entropy, W_gate, W_ent, b_ent, temperature):
    x2 = x.reshape(_TOKENS, _D)
    went = W_ent.reshape(1, _E)
    bent = b_ent.reshape(1, _E)
    logits, logitsT = _gate_logits(x2, W_gate, went, bent, entropy, temperature)
    weights2, experts2 = _topk_call(logitsT)
    return (
        weights2.T.reshape(_B, _T, 2),
        experts2.T.reshape(_B, _T, 2),
        logits.reshape(_B, _T, _E),
    )
